# initial kernel scaffold (unmeasured)
import jax
import jax.numpy as jnp
from jax import lax
from jax.experimental import pallas as pl
from jax.experimental.pallas import tpu as pltpu


def kernel(x, router, W1, W2):
    T_loc, D = x.shape
    E_loc = router.shape[1]
    F = W1.shape[2]
    T = 2 * T_loc

    def body(x_ref, r_ref, w1_ref, w2_ref, out_ref,
             xg_ref, r_oth_ref, wfull_ref, wsend_ref, psend_ref, precv_ref,
             send_sems, recv_sems, acc_ref):
        my_x = lax.axis_index("x")
        my_y = lax.axis_index("y")
        my_z = lax.axis_index("z")
        partner = (my_x, 1 - my_y, my_z)

        barrier_sem = pltpu.get_barrier_semaphore()
        pl.semaphore_signal(barrier_sem, inc=1, device_id=partner,
                            device_id_type=pl.DeviceIdType.MESH)
        pl.semaphore_wait(barrier_sem, 1)

        row0 = my_y * T_loc

        xg_ref[pl.ds(row0, T_loc), :] = x_ref[:, :].astype(jnp.bfloat16)
        x_rdma = pltpu.make_async_remote_copy(
            src_ref=xg_ref.at[pl.ds(row0, T_loc), :],
            dst_ref=xg_ref.at[pl.ds(row0, T_loc), :],
            send_sem=send_sems.at[0],
            recv_sem=recv_sems.at[0],
            device_id=partner,
            device_id_type=pl.DeviceIdType.MESH,
        )
        x_rdma.start()

        r_rdma = pltpu.make_async_remote_copy(
            src_ref=r_ref,
            dst_ref=r_oth_ref,
            send_sem=send_sems.at[1],
            recv_sem=recv_sems.at[1],
            device_id=partner,
            device_id_type=pl.DeviceIdType.MESH,
        )
        r_rdma.start()
        r_rdma.wait_recv()

        xl = x_ref[:, :]
        g_loc = jnp.dot(xl, r_ref[:, :], precision=lax.Precision.HIGHEST,
                        preferred_element_type=jnp.float32)
        g_oth = jnp.dot(xl, r_oth_ref[:, :], precision=lax.Precision.HIGHEST,
                        preferred_element_type=jnp.float32)
        g = jnp.where(my_y == 0,
                      jnp.concatenate([g_loc, g_oth], axis=1),
                      jnp.concatenate([g_oth, g_loc], axis=1))

        neg = jnp.float32(-1e30)
        m1 = jnp.max(g, axis=1, keepdims=True)
        is1 = g == m1
        m2 = jnp.max(jnp.where(is1, neg, g), axis=1, keepdims=True)
        is2 = jnp.logical_and(g == m2, jnp.logical_not(is1))
        e12 = jnp.exp(m2 - m1)
        denom = 1.0 + e12
        w8 = jnp.where(is1, 1.0 / denom, 0.0) + jnp.where(is2, e12 / denom, 0.0)

        w_lo = w8[:, :E_loc]
        w_hi = w8[:, E_loc:]
        w_mine = jnp.where(my_y == 0, w_lo, w_hi)
        w_theirs = jnp.where(my_y == 0, w_hi, w_lo)
        wfull_ref[pl.ds(row0, T_loc), :] = w_mine
        wsend_ref[:, :] = w_theirs
        w_rdma = pltpu.make_async_remote_copy(
            src_ref=wsend_ref,
            dst_ref=wfull_ref.at[pl.ds(row0, T_loc), :],
            send_sem=send_sems.at[2],
            recv_sem=recv_sems.at[2],
            device_id=partner,
            device_id_type=pl.DeviceIdType.MESH,
        )
        w_rdma.start()

        x_rdma.wait_recv()
        w_rdma.wait_recv()

        n_chunks = 2
        Tc = T // n_chunks
        for e in range(E_loc):
            w1e = w1_ref[e, :, :].astype(jnp.bfloat16)
            w2e = w2_ref[e, :, :].astype(jnp.bfloat16)
            for c in range(n_chunks):
                xc = xg_ref[c * Tc:(c + 1) * Tc, :]
                h = jnp.dot(xc, w1e, preferred_element_type=jnp.float32)
                h = jnp.maximum(h, 0.0).astype(jnp.bfloat16)
                oe = jnp.dot(h, w2e, preferred_element_type=jnp.float32)
                oe = oe * wfull_ref[c * Tc:(c + 1) * Tc, e:e + 1]
                if e == 0:
                    acc_ref[c * Tc:(c + 1) * Tc, :] = oe
                else:
                    acc_ref[c * Tc:(c + 1) * Tc, :] += oe

        acc_lo = acc_ref[:T_loc, :]
        acc_hi = acc_ref[T_loc:, :]
        mine = jnp.where(my_y == 0, acc_lo, acc_hi)
        theirs = jnp.where(my_y == 0, acc_hi, acc_lo)
        psend_ref[:, :] = theirs.astype(jnp.bfloat16)
        p_rdma = pltpu.make_async_remote_copy(
            src_ref=psend_ref,
            dst_ref=precv_ref,
            send_sem=send_sems.at[3],
            recv_sem=recv_sems.at[3],
            device_id=partner,
            device_id_type=pl.DeviceIdType.MESH,
        )
        p_rdma.start()
        p_rdma.wait_recv()
        out_ref[:, :] = mine + precv_ref[:, :].astype(jnp.float32)

        x_rdma.wait_send()
        r_rdma.wait_send()
        w_rdma.wait_send()
        p_rdma.wait_send()

    return pl.pallas_call(
        body,
        out_shape=jax.ShapeDtypeStruct((T_loc, D), jnp.float32),
        in_specs=[pl.BlockSpec(memory_space=pltpu.VMEM)] * 4,
        out_specs=pl.BlockSpec(memory_space=pltpu.VMEM),
        scratch_shapes=[
            pltpu.VMEM((T, D), jnp.bfloat16),
            pltpu.VMEM((D, E_loc), jnp.float32),
            pltpu.VMEM((T, E_loc), jnp.float32),
            pltpu.VMEM((T_loc, E_loc), jnp.float32),
            pltpu.VMEM((T_loc, D), jnp.bfloat16),
            pltpu.VMEM((T_loc, D), jnp.bfloat16),
            pltpu.SemaphoreType.DMA((4,)),
            pltpu.SemaphoreType.DMA((4,)),
            pltpu.VMEM((T, D), jnp.float32),
        ],
        compiler_params=pltpu.CompilerParams(
            collective_id=0,
            vmem_limit_bytes=128 * 1024 * 1024,
        ),
    )(x, router, W1, W2)


# baseline (device time: 131274 ns/iter reference)
import jax
import jax.numpy as jnp
from jax import lax
from jax.experimental import pallas as pl
from jax.experimental.pallas import tpu as pltpu


def kernel(x, router, W1, W2):
    T_loc, D = x.shape
    E_loc = router.shape[1]
    F = W1.shape[2]
    T = 2 * T_loc

    def body(x_ref, r_ref, w1_ref, w2_ref, out_ref,
             xg_ref, r_oth_ref, wfull_ref, wsend_ref, psend_ref, precv_ref,
             send_sems, recv_sems, acc_ref):
        my_x = lax.axis_index("x")
        my_y = lax.axis_index("y")
        my_z = lax.axis_index("z")
        partner = (my_x, 1 - my_y, my_z)

        barrier_sem = pltpu.get_barrier_semaphore()
        pl.semaphore_signal(barrier_sem, inc=1, device_id=partner,
                            device_id_type=pl.DeviceIdType.MESH)
        pl.semaphore_wait(barrier_sem, 1)

        row0 = my_y * T_loc

        xg_ref[pl.ds(row0, T_loc), :] = x_ref[:, :].astype(jnp.bfloat16)
        x_rdma = pltpu.make_async_remote_copy(
            src_ref=xg_ref.at[pl.ds(row0, T_loc), :],
            dst_ref=xg_ref.at[pl.ds(row0, T_loc), :],
            send_sem=send_sems.at[0],
            recv_sem=recv_sems.at[0],
            device_id=partner,
            device_id_type=pl.DeviceIdType.MESH,
        )
        x_rdma.start()

        r_rdma = pltpu.make_async_remote_copy(
            src_ref=r_ref,
            dst_ref=r_oth_ref,
            send_sem=send_sems.at[1],
            recv_sem=recv_sems.at[1],
            device_id=partner,
            device_id_type=pl.DeviceIdType.MESH,
        )
        r_rdma.start()
        r_rdma.wait_recv()

        xl = x_ref[:, :]
        g_loc = jnp.dot(xl, r_ref[:, :], precision=lax.Precision.HIGHEST,
                        preferred_element_type=jnp.float32)
        g_oth = jnp.dot(xl, r_oth_ref[:, :], precision=lax.Precision.HIGHEST,
                        preferred_element_type=jnp.float32)
        g = jnp.where(my_y == 0,
                      jnp.concatenate([g_loc, g_oth], axis=1),
                      jnp.concatenate([g_oth, g_loc], axis=1))

        neg = jnp.float32(-1e30)
        m1 = jnp.max(g, axis=1, keepdims=True)
        is1 = g == m1
        m2 = jnp.max(jnp.where(is1, neg, g), axis=1, keepdims=True)
        is2 = jnp.logical_and(g == m2, jnp.logical_not(is1))
        e12 = jnp.exp(m2 - m1)
        denom = 1.0 + e12
        w8 = jnp.where(is1, 1.0 / denom, 0.0) + jnp.where(is2, e12 / denom, 0.0)

        w_lo = w8[:, :E_loc]
        w_hi = w8[:, E_loc:]
        w_mine = jnp.where(my_y == 0, w_lo, w_hi)
        w_theirs = jnp.where(my_y == 0, w_hi, w_lo)
        wfull_ref[pl.ds(row0, T_loc), :] = w_mine
        wsend_ref[:, :] = w_theirs
        w_rdma = pltpu.make_async_remote_copy(
            src_ref=wsend_ref,
            dst_ref=wfull_ref.at[pl.ds(row0, T_loc), :],
            send_sem=send_sems.at[2],
            recv_sem=recv_sems.at[2],
            device_id=partner,
            device_id_type=pl.DeviceIdType.MESH,
        )
        w_rdma.start()

        x_rdma.wait_recv()
        w_rdma.wait_recv()

        n_chunks = 2
        Tc = T // n_chunks
        for e in range(E_loc):
            for c in range(n_chunks):
                xc = xg_ref[c * Tc:(c + 1) * Tc, :]
                h = jnp.dot(xc, w1_ref[e, :, :],
                            preferred_element_type=jnp.float32)
                h = jnp.maximum(h, 0.0).astype(jnp.bfloat16)
                oe = jnp.dot(h, w2_ref[e, :, :],
                             preferred_element_type=jnp.float32)
                oe = oe * wfull_ref[c * Tc:(c + 1) * Tc, e:e + 1]
                if e == 0:
                    acc_ref[c * Tc:(c + 1) * Tc, :] = oe
                else:
                    acc_ref[c * Tc:(c + 1) * Tc, :] += oe

        acc_lo = acc_ref[:T_loc, :]
        acc_hi = acc_ref[T_loc:, :]
        mine = jnp.where(my_y == 0, acc_lo, acc_hi)
        theirs = jnp.where(my_y == 0, acc_hi, acc_lo)
        psend_ref[:, :] = theirs.astype(jnp.bfloat16)
        p_rdma = pltpu.make_async_remote_copy(
            src_ref=psend_ref,
            dst_ref=precv_ref,
            send_sem=send_sems.at[3],
            recv_sem=recv_sems.at[3],
            device_id=partner,
            device_id_type=pl.DeviceIdType.MESH,
        )
        p_rdma.start()
        p_rdma.wait_recv()
        out_ref[:, :] = mine + precv_ref[:, :].astype(jnp.float32)

        x_rdma.wait_send()
        r_rdma.wait_send()
        w_rdma.wait_send()
        p_rdma.wait_send()

    return pl.pallas_call(
        body,
        out_shape=jax.ShapeDtypeStruct((T_loc, D), jnp.float32),
        in_specs=[pl.BlockSpec(memory_space=pltpu.VMEM)] * 4,
        out_specs=pl.BlockSpec(memory_space=pltpu.VMEM),
        scratch_shapes=[
            pltpu.VMEM((T, D), jnp.bfloat16),
            pltpu.VMEM((D, E_loc), jnp.float32),
            pltpu.VMEM((T, E_loc), jnp.float32),
            pltpu.VMEM((T_loc, E_loc), jnp.float32),
            pltpu.VMEM((T_loc, D), jnp.bfloat16),
            pltpu.VMEM((T_loc, D), jnp.bfloat16),
            pltpu.SemaphoreType.DMA((4,)),
            pltpu.SemaphoreType.DMA((4,)),
            pltpu.VMEM((T, D), jnp.float32),
        ],
        compiler_params=pltpu.CompilerParams(
            collective_id=0,
            vmem_limit_bytes=128 * 1024 * 1024,
        ),
    )(x, router, W1.astype(jnp.bfloat16), W2.astype(jnp.bfloat16))


# device time: 93089 ns/iter; 1.4102x vs baseline; 1.4102x over previous
import jax
import jax.numpy as jnp
from jax import lax
from jax.experimental import pallas as pl
from jax.experimental.pallas import tpu as pltpu


def kernel(x, router, W1, W2):
    T_loc, D = x.shape
    E_loc = router.shape[1]
    F = W1.shape[2]
    T = 2 * T_loc

    def body(x_ref, r_ref, w1_ref, w2_ref, out_ref,
             xg_ref, r_oth_ref, wfull_ref, wsend_ref, psend_ref, precv_ref,
             pacc_ref, w1s_ref, w2s_ref,
             send_sems, recv_sems, wdma_sems):
        my_x = lax.axis_index("x")
        my_y = lax.axis_index("y")
        my_z = lax.axis_index("z")
        partner = (my_x, 1 - my_y, my_z)

        def wdma(e):
            return (
                pltpu.make_async_copy(w1_ref.at[e], w1s_ref, wdma_sems.at[0]),
                pltpu.make_async_copy(w2_ref.at[e], w2s_ref, wdma_sems.at[1]),
            )

        c1, c2 = wdma(0)
        c1.start()
        c2.start()

        barrier_sem = pltpu.get_barrier_semaphore()
        pl.semaphore_signal(barrier_sem, inc=1, device_id=partner,
                            device_id_type=pl.DeviceIdType.MESH)
        pl.semaphore_wait(barrier_sem, 1)

        row0 = my_y * T_loc
        prow0 = (1 - my_y) * T_loc

        xg_ref[pl.ds(row0, T_loc), :] = x_ref[:, :].astype(jnp.bfloat16)
        x_rdma = pltpu.make_async_remote_copy(
            src_ref=xg_ref.at[pl.ds(row0, T_loc), :],
            dst_ref=xg_ref.at[pl.ds(row0, T_loc), :],
            send_sem=send_sems.at[0],
            recv_sem=recv_sems.at[0],
            device_id=partner,
            device_id_type=pl.DeviceIdType.MESH,
        )
        x_rdma.start()

        r_rdma = pltpu.make_async_remote_copy(
            src_ref=r_ref,
            dst_ref=r_oth_ref,
            send_sem=send_sems.at[1],
            recv_sem=recv_sems.at[1],
            device_id=partner,
            device_id_type=pl.DeviceIdType.MESH,
        )
        r_rdma.start()
        r_rdma.wait_recv()

        xl = x_ref[:, :]
        g_loc = jnp.dot(xl, r_ref[:, :], precision=lax.Precision.HIGHEST,
                        preferred_element_type=jnp.float32)
        g_oth = jnp.dot(xl, r_oth_ref[:, :], precision=lax.Precision.HIGHEST,
                        preferred_element_type=jnp.float32)
        g = jnp.where(my_y == 0,
                      jnp.concatenate([g_loc, g_oth], axis=1),
                      jnp.concatenate([g_oth, g_loc], axis=1))

        neg = jnp.float32(-1e30)
        m1 = jnp.max(g, axis=1, keepdims=True)
        is1 = g == m1
        m2 = jnp.max(jnp.where(is1, neg, g), axis=1, keepdims=True)
        is2 = jnp.logical_and(g == m2, jnp.logical_not(is1))
        e12 = jnp.exp(m2 - m1)
        denom = 1.0 + e12
        w8 = jnp.where(is1, 1.0 / denom, 0.0) + jnp.where(is2, e12 / denom, 0.0)

        w_lo = w8[:, :E_loc]
        w_hi = w8[:, E_loc:]
        w_mine = jnp.where(my_y == 0, w_lo, w_hi)
        w_theirs = jnp.where(my_y == 0, w_hi, w_lo)
        wfull_ref[pl.ds(row0, T_loc), :] = w_mine
        wsend_ref[:, :] = w_theirs
        w_rdma = pltpu.make_async_remote_copy(
            src_ref=wsend_ref,
            dst_ref=wfull_ref.at[pl.ds(row0, T_loc), :],
            send_sem=send_sems.at[2],
            recv_sem=recv_sems.at[2],
            device_id=partner,
            device_id_type=pl.DeviceIdType.MESH,
        )
        w_rdma.start()

        def half(e, off, w1bf, w2bf):
            xc = xg_ref[pl.ds(off, T_loc), :]
            h = jnp.dot(xc, w1bf, preferred_element_type=jnp.float32)
            h = jnp.maximum(h, 0.0).astype(jnp.bfloat16)
            oe = jnp.dot(h, w2bf, preferred_element_type=jnp.float32)
            wcol = wfull_ref[pl.ds(off, T_loc), :][:, e:e + 1]
            return oe * wcol

        p_rdma = pltpu.make_async_remote_copy(
            src_ref=psend_ref,
            dst_ref=precv_ref,
            send_sem=send_sems.at[3],
            recv_sem=recv_sems.at[3],
            device_id=partner,
            device_id_type=pl.DeviceIdType.MESH,
        )

        for e in range(E_loc):
            c1, c2 = wdma(e)
            c1.wait()
            c2.wait()
            w1bf = w1s_ref[:, :].astype(jnp.bfloat16)
            w2bf = w2s_ref[:, :].astype(jnp.bfloat16)

            if e < E_loc - 1:
                first_off, second_off = row0, prow0
            else:
                first_off, second_off = prow0, row0

            o_first = half(e, first_off, w1bf, w2bf)
            if e < E_loc - 1:
                if e == 0:
                    out_ref[:, :] = o_first
                else:
                    out_ref[:, :] += o_first
            else:
                psend_ref[:, :] = (pacc_ref[:, :] + o_first).astype(jnp.bfloat16)
                p_rdma.start()

            if e < E_loc - 1:
                n1, n2 = wdma(e + 1)
                n1.start()
                n2.start()

            if e == 0:
                x_rdma.wait_recv()
                w_rdma.wait_recv()

            o_second = half(e, second_off, w1bf, w2bf)
            if e < E_loc - 1:
                if e == 0:
                    pacc_ref[:, :] = o_second
                else:
                    pacc_ref[:, :] += o_second
            else:
                out_ref[:, :] += o_second

        p_rdma.wait_recv()
        out_ref[:, :] += precv_ref[:, :].astype(jnp.float32)

        x_rdma.wait_send()
        r_rdma.wait_send()
        w_rdma.wait_send()
        p_rdma.wait_send()

    return pl.pallas_call(
        body,
        out_shape=jax.ShapeDtypeStruct((T_loc, D), jnp.float32),
        in_specs=[
            pl.BlockSpec(memory_space=pltpu.VMEM),
            pl.BlockSpec(memory_space=pltpu.VMEM),
            pl.BlockSpec(memory_space=pl.ANY),
            pl.BlockSpec(memory_space=pl.ANY),
        ],
        out_specs=pl.BlockSpec(memory_space=pltpu.VMEM),
        scratch_shapes=[
            pltpu.VMEM((T, D), jnp.bfloat16),
            pltpu.VMEM((D, E_loc), jnp.float32),
            pltpu.VMEM((T, E_loc), jnp.float32),
            pltpu.VMEM((T_loc, E_loc), jnp.float32),
            pltpu.VMEM((T_loc, D), jnp.bfloat16),
            pltpu.VMEM((T_loc, D), jnp.bfloat16),
            pltpu.VMEM((T_loc, D), jnp.float32),
            pltpu.VMEM((D, F), jnp.float32),
            pltpu.VMEM((F, D), jnp.float32),
            pltpu.SemaphoreType.DMA((4,)),
            pltpu.SemaphoreType.DMA((4,)),
            pltpu.SemaphoreType.DMA((2,)),
        ],
        compiler_params=pltpu.CompilerParams(
            collective_id=0,
            vmem_limit_bytes=128 * 1024 * 1024,
        ),
    )(x, router, W1, W2)
